# baseline (speedup 1.0000x reference)
import functools

import jax
import jax.numpy as jnp
from jax import lax
from jax.experimental import pallas as pl
from jax.experimental.pallas import tpu as pltpu
from jax.experimental.pallas import tpu_sc as plsc

B, L, H_DIM = 2, 4096, 2048
NH, NP = 16, 5
MIN_ACTIVE = 2
FLOOR_START = 0.05

M = B * L
N = 2 * H_DIM
TM = 2048
TN = 2048

NUM_WORKERS = 32
TOK_W = M // NUM_WORKERS
CH = TOK_W * NH * NP


def _mlp_kernel(x_ref, w1_ref, b1_ref, w2_ref, b2_ref, out_ref):
    n = pl.program_id(1)
    h = (
        jnp.dot(x_ref[...], w1_ref[...], preferred_element_type=jnp.float32)
        + b1_ref[...]
    )
    h = 0.5 * h * (1.0 + jax.lax.erf(h * 0.7071067811865476))
    contrib = jnp.dot(
        h.astype(jnp.bfloat16), w2_ref[...], preferred_element_type=jnp.float32
    )

    @pl.when(n == 0)
    def _():
        out_ref[...] = contrib + b2_ref[...]

    @pl.when(n != 0)
    def _():
        out_ref[...] += contrib


def _sc_epilogue(logits_hbm, invt_hbm, epsb_hbm, out_hbm,
                 in_v, out_v, invt_v, epsb_v):
    wid = lax.axis_index("s") * 2 + lax.axis_index("c")
    base = wid * CH
    pltpu.sync_copy(logits_hbm.at[pl.ds(base, CH)], in_v)
    pltpu.sync_copy(invt_hbm, invt_v)
    pltpu.sync_copy(epsb_hbm, epsb_v)
    invt = invt_v[...]
    eb = [epsb_v[p, :] for p in range(NP)]
    lane5 = lax.iota(jnp.int32, 16) * 5

    def body(t, _):
        tbase = t * (NH * NP)
        idx = [lane5 + (tbase + p) for p in range(NP)]
        z = [plsc.load_gather(in_v, [idx[p]]) * invt for p in range(NP)]
        zmax = jnp.maximum(jnp.maximum(jnp.maximum(z[0], z[1]),
                                       jnp.maximum(z[2], z[3])), z[4])
        e = [jnp.exp(z[p] - zmax) for p in range(NP)]
        s = (e[0] + e[1]) + (e[2] + e[3]) + e[4]
        r = 1.0 / s
        p_ = [e[p] * r for p in range(NP)]
        eps_scale = FLOOR_START * (1.0 - r)
        eps = [eps_scale * eb[p] for p in range(NP)]
        eps_sum = ((eps[0] + eps[1]) + (eps[2] + eps[3])) + eps[4]
        keep = 1.0 - eps_sum
        q = [jnp.minimum(jnp.maximum(p_[p] * keep + eps[p], 1e-9), 1.0)
             for p in range(NP)]
        cnt = sum((q[p] > 1e-6).astype(jnp.int32) for p in range(NP))
        insufficient = cnt < MIN_ACTIVE
        best1 = q[0]
        idx1 = jnp.zeros((16,), jnp.int32)
        for p in range(1, NP):
            better = q[p] > best1
            best1 = jnp.where(better, q[p], best1)
            idx1 = jnp.where(better, p, idx1)
        best2 = jnp.full((16,), -1.0, jnp.float32)
        idx2 = jnp.zeros((16,), jnp.int32)
        for p in range(NP):
            cand = jnp.where(idx1 == p, -1.0, q[p])
            better = cand > best2
            best2 = jnp.where(better, cand, best2)
            idx2 = jnp.where(better, p, idx2)
        for p in range(NP):
            on_top = jnp.logical_or(idx1 == p, idx2 == p)
            uni = jnp.where(on_top, 1.0 / MIN_ACTIVE, 0.0)
            res = jnp.where(insufficient, uni, q[p])
            plsc.store_scatter(out_v, [idx[p]], res)
        return _

    lax.fori_loop(0, TOK_W, body, None)
    pltpu.sync_copy(out_v, out_hbm.at[pl.ds(base, CH)])


@jax.jit
def kernel(x, W1, b1, W2, b2, gate_log_temp, gate_eps_logit):
    x2 = x.reshape(M, H_DIM).astype(jnp.bfloat16)
    W1b = W1.astype(jnp.bfloat16)
    W2b = W2.astype(jnp.bfloat16)
    logits = pl.pallas_call(
        _mlp_kernel,
        grid=(M // TM, N // TN),
        in_specs=[
            pl.BlockSpec((TM, H_DIM), lambda m, n: (m, 0)),
            pl.BlockSpec((H_DIM, TN), lambda m, n: (0, n)),
            pl.BlockSpec((1, TN), lambda m, n: (0, n)),
            pl.BlockSpec((TN, NH * NP), lambda m, n: (n, 0)),
            pl.BlockSpec((1, NH * NP), lambda m, n: (0, 0)),
        ],
        out_specs=pl.BlockSpec((TM, NH * NP), lambda m, n: (m, 0)),
        out_shape=jax.ShapeDtypeStruct((M, NH * NP), jnp.float32),
        compiler_params=pltpu.CompilerParams(
            dimension_semantics=("parallel", "arbitrary")
        ),
    )(x2, W1b, b1.reshape(1, N), W2b, b2.reshape(1, NH * NP))

    inv_temp = jnp.exp(-gate_log_temp)
    eps_base_t = jax.nn.sigmoid(gate_eps_logit).T

    mesh = plsc.VectorSubcoreMesh(core_axis_name="c", subcore_axis_name="s")
    sc_epi = functools.partial(
        pl.kernel,
        mesh=mesh,
        compiler_params=pltpu.CompilerParams(needs_layout_passes=False),
        out_type=jax.ShapeDtypeStruct((M * NH * NP,), jnp.float32),
        scratch_types=[
            pltpu.VMEM((CH,), jnp.float32),
            pltpu.VMEM((CH,), jnp.float32),
            pltpu.VMEM((NH,), jnp.float32),
            pltpu.VMEM((NP, NH), jnp.float32),
        ],
    )(_sc_epilogue)
    probs = sc_epi(logits.reshape(M * NH * NP), inv_temp, eps_base_t)
    return probs.reshape(B, L, NH, NP)


# speedup vs baseline: 1.0910x; 1.0910x over previous
import functools

import jax
import jax.numpy as jnp
from jax import lax
from jax.experimental import pallas as pl
from jax.experimental.pallas import tpu as pltpu
from jax.experimental.pallas import tpu_sc as plsc

B, L, H_DIM = 2, 4096, 2048
NH, NP = 16, 5
MIN_ACTIVE = 2
FLOOR_START = 0.05

M = B * L
N = 2 * H_DIM
TM = 2048
TN = 1024

NUM_WORKERS = 32
TOK_W = M // NUM_WORKERS
CH = TOK_W * NH * NP


def _mlp_kernel(x_ref, w1_ref, b1_ref, w2_ref, b2_ref, out_ref):
    n = pl.program_id(1)
    h = (
        jnp.dot(
            x_ref[...].astype(jnp.bfloat16),
            w1_ref[...],
            preferred_element_type=jnp.float32,
        )
        + b1_ref[...]
    )
    h = 0.5 * h * (1.0 + jax.lax.erf(h * 0.7071067811865476))
    contrib = jnp.dot(
        h.astype(jnp.bfloat16), w2_ref[...], preferred_element_type=jnp.float32
    )

    @pl.when(n == 0)
    def _():
        out_ref[...] = contrib + b2_ref[...]

    @pl.when(n != 0)
    def _():
        out_ref[...] += contrib


def _sc_epilogue(logits_hbm, invt_hbm, epsb_hbm, out_hbm,
                 in_v, out_v, invt_v, epsb_v):
    wid = lax.axis_index("s") * 2 + lax.axis_index("c")
    base = wid * CH
    pltpu.sync_copy(logits_hbm.at[pl.ds(base, CH)], in_v)
    pltpu.sync_copy(invt_hbm, invt_v)
    pltpu.sync_copy(epsb_hbm, epsb_v)
    invt = invt_v[...]
    eb = [epsb_v[p, :] for p in range(NP)]
    lane5 = lax.iota(jnp.int32, 16) * 5

    def body(t, _):
        tbase = t * (NH * NP)
        idx = [lane5 + (tbase + p) for p in range(NP)]
        z = [plsc.load_gather(in_v, [idx[p]]) * invt for p in range(NP)]
        zmax = jnp.maximum(jnp.maximum(jnp.maximum(z[0], z[1]),
                                       jnp.maximum(z[2], z[3])), z[4])
        e = [jnp.exp(z[p] - zmax) for p in range(NP)]
        s = (e[0] + e[1]) + (e[2] + e[3]) + e[4]
        r = 1.0 / s
        p_ = [e[p] * r for p in range(NP)]
        eps_scale = FLOOR_START * (1.0 - r)
        eps = [eps_scale * eb[p] for p in range(NP)]
        eps_sum = ((eps[0] + eps[1]) + (eps[2] + eps[3])) + eps[4]
        keep = 1.0 - eps_sum
        q = [jnp.minimum(jnp.maximum(p_[p] * keep + eps[p], 1e-9), 1.0)
             for p in range(NP)]
        cnt = sum((q[p] > 1e-6).astype(jnp.int32) for p in range(NP))
        insufficient = cnt < MIN_ACTIVE
        best1 = q[0]
        idx1 = jnp.zeros((16,), jnp.int32)
        for p in range(1, NP):
            better = q[p] > best1
            best1 = jnp.where(better, q[p], best1)
            idx1 = jnp.where(better, p, idx1)
        best2 = jnp.full((16,), -1.0, jnp.float32)
        idx2 = jnp.zeros((16,), jnp.int32)
        for p in range(NP):
            cand = jnp.where(idx1 == p, -1.0, q[p])
            better = cand > best2
            best2 = jnp.where(better, cand, best2)
            idx2 = jnp.where(better, p, idx2)
        for p in range(NP):
            on_top = jnp.logical_or(idx1 == p, idx2 == p)
            uni = jnp.where(on_top, 1.0 / MIN_ACTIVE, 0.0)
            res = jnp.where(insufficient, uni, q[p])
            plsc.store_scatter(out_v, [idx[p]], res)
        return _

    lax.fori_loop(0, TOK_W, body, None)
    pltpu.sync_copy(out_v, out_hbm.at[pl.ds(base, CH)])


@jax.jit
def kernel(x, W1, b1, W2, b2, gate_log_temp, gate_eps_logit):
    x2 = x.reshape(M, H_DIM)
    W1b = W1.astype(jnp.bfloat16)
    W2b = W2.astype(jnp.bfloat16)
    logits = pl.pallas_call(
        _mlp_kernel,
        grid=(M // TM, N // TN),
        in_specs=[
            pl.BlockSpec((TM, H_DIM), lambda m, n: (m, 0)),
            pl.BlockSpec((H_DIM, TN), lambda m, n: (0, n)),
            pl.BlockSpec((1, TN), lambda m, n: (0, n)),
            pl.BlockSpec((TN, NH * NP), lambda m, n: (n, 0)),
            pl.BlockSpec((1, NH * NP), lambda m, n: (0, 0)),
        ],
        out_specs=pl.BlockSpec((TM, NH * NP), lambda m, n: (m, 0)),
        out_shape=jax.ShapeDtypeStruct((M, NH * NP), jnp.float32),
        compiler_params=pltpu.CompilerParams(
            dimension_semantics=("parallel", "arbitrary")
        ),
    )(x2, W1b, b1.reshape(1, N), W2b, b2.reshape(1, NH * NP))

    inv_temp = jnp.exp(-gate_log_temp)
    eps_base_t = jax.nn.sigmoid(gate_eps_logit).T

    mesh = plsc.VectorSubcoreMesh(core_axis_name="c", subcore_axis_name="s")
    sc_epi = functools.partial(
        pl.kernel,
        mesh=mesh,
        compiler_params=pltpu.CompilerParams(needs_layout_passes=False),
        out_type=jax.ShapeDtypeStruct((M * NH * NP,), jnp.float32),
        scratch_types=[
            pltpu.VMEM((CH,), jnp.float32),
            pltpu.VMEM((CH,), jnp.float32),
            pltpu.VMEM((NH,), jnp.float32),
            pltpu.VMEM((NP, NH), jnp.float32),
        ],
    )(_sc_epilogue)
    probs = sc_epi(logits.reshape(M * NH * NP), inv_temp, eps_base_t)
    return probs.reshape(B, L, NH, NP)


# speedup vs baseline: 1.0990x; 1.0073x over previous
import functools

import jax
import jax.numpy as jnp
from jax import lax
from jax.experimental import pallas as pl
from jax.experimental.pallas import tpu as pltpu
from jax.experimental.pallas import tpu_sc as plsc

B, L, H_DIM = 2, 4096, 2048
NH, NP = 16, 5
MIN_ACTIVE = 2
FLOOR_START = 0.05

M = B * L
N = 2 * H_DIM
TM = 1024
TN = 2048

NUM_WORKERS = 32
TOK_W = M // NUM_WORKERS
CH = TOK_W * NH * NP


def _mlp_kernel(x_ref, w1_ref, b1_ref, w2_ref, b2_ref, out_ref):
    n = pl.program_id(1)
    h = (
        jnp.dot(
            x_ref[...].astype(jnp.bfloat16),
            w1_ref[...],
            preferred_element_type=jnp.float32,
        )
        + b1_ref[...]
    )
    h = 0.5 * h * (1.0 + jax.lax.erf(h * 0.7071067811865476))
    contrib = jnp.dot(
        h.astype(jnp.bfloat16), w2_ref[...], preferred_element_type=jnp.float32
    )

    @pl.when(n == 0)
    def _():
        out_ref[...] = contrib + b2_ref[...]

    @pl.when(n != 0)
    def _():
        out_ref[...] += contrib


def _sc_epilogue(logits_hbm, invt_hbm, epsb_hbm, out_hbm,
                 in_v, out_v, invt_v, epsb_v):
    wid = lax.axis_index("s") * 2 + lax.axis_index("c")
    base = wid * CH
    pltpu.sync_copy(logits_hbm.at[pl.ds(base, CH)], in_v)
    pltpu.sync_copy(invt_hbm, invt_v)
    pltpu.sync_copy(epsb_hbm, epsb_v)
    invt = invt_v[...]
    eb = [epsb_v[p, :] for p in range(NP)]
    lane5 = lax.iota(jnp.int32, 16) * 5

    def body(t, _):
        tbase = t * (NH * NP)
        idx = [lane5 + (tbase + p) for p in range(NP)]
        z = [plsc.load_gather(in_v, [idx[p]]) * invt for p in range(NP)]
        zmax = jnp.maximum(jnp.maximum(jnp.maximum(z[0], z[1]),
                                       jnp.maximum(z[2], z[3])), z[4])
        e = [jnp.exp(z[p] - zmax) for p in range(NP)]
        s = (e[0] + e[1]) + (e[2] + e[3]) + e[4]
        r = 1.0 / s
        p_ = [e[p] * r for p in range(NP)]
        eps_scale = FLOOR_START * (1.0 - r)
        eps = [eps_scale * eb[p] for p in range(NP)]
        eps_sum = ((eps[0] + eps[1]) + (eps[2] + eps[3])) + eps[4]
        keep = 1.0 - eps_sum
        q = [jnp.minimum(jnp.maximum(p_[p] * keep + eps[p], 1e-9), 1.0)
             for p in range(NP)]
        cnt = sum((q[p] > 1e-6).astype(jnp.int32) for p in range(NP))
        insufficient = cnt < MIN_ACTIVE
        best1 = q[0]
        idx1 = jnp.zeros((16,), jnp.int32)
        for p in range(1, NP):
            better = q[p] > best1
            best1 = jnp.where(better, q[p], best1)
            idx1 = jnp.where(better, p, idx1)
        best2 = jnp.full((16,), -1.0, jnp.float32)
        idx2 = jnp.zeros((16,), jnp.int32)
        for p in range(NP):
            cand = jnp.where(idx1 == p, -1.0, q[p])
            better = cand > best2
            best2 = jnp.where(better, cand, best2)
            idx2 = jnp.where(better, p, idx2)
        for p in range(NP):
            on_top = jnp.logical_or(idx1 == p, idx2 == p)
            uni = jnp.where(on_top, 1.0 / MIN_ACTIVE, 0.0)
            res = jnp.where(insufficient, uni, q[p])
            plsc.store_scatter(out_v, [idx[p]], res)
        return _

    lax.fori_loop(0, TOK_W, body, None)
    pltpu.sync_copy(out_v, out_hbm.at[pl.ds(base, CH)])


@jax.jit
def kernel(x, W1, b1, W2, b2, gate_log_temp, gate_eps_logit):
    x2 = x.reshape(M, H_DIM)
    W1b = W1.astype(jnp.bfloat16)
    W2b = W2.astype(jnp.bfloat16)
    logits = pl.pallas_call(
        _mlp_kernel,
        grid=(M // TM, N // TN),
        in_specs=[
            pl.BlockSpec((TM, H_DIM), lambda m, n: (m, 0)),
            pl.BlockSpec((H_DIM, TN), lambda m, n: (0, n)),
            pl.BlockSpec((1, TN), lambda m, n: (0, n)),
            pl.BlockSpec((TN, NH * NP), lambda m, n: (n, 0)),
            pl.BlockSpec((1, NH * NP), lambda m, n: (0, 0)),
        ],
        out_specs=pl.BlockSpec((TM, NH * NP), lambda m, n: (m, 0)),
        out_shape=jax.ShapeDtypeStruct((M, NH * NP), jnp.float32),
        compiler_params=pltpu.CompilerParams(
            dimension_semantics=("parallel", "arbitrary")
        ),
    )(x2, W1b, b1.reshape(1, N), W2b, b2.reshape(1, NH * NP))

    inv_temp = jnp.exp(-gate_log_temp)
    eps_base_t = jax.nn.sigmoid(gate_eps_logit).T

    mesh = plsc.VectorSubcoreMesh(core_axis_name="c", subcore_axis_name="s")
    sc_epi = functools.partial(
        pl.kernel,
        mesh=mesh,
        compiler_params=pltpu.CompilerParams(needs_layout_passes=False),
        out_type=jax.ShapeDtypeStruct((M * NH * NP,), jnp.float32),
        scratch_types=[
            pltpu.VMEM((CH,), jnp.float32),
            pltpu.VMEM((CH,), jnp.float32),
            pltpu.VMEM((NH,), jnp.float32),
            pltpu.VMEM((NP, NH), jnp.float32),
        ],
    )(_sc_epilogue)
    probs = sc_epi(logits.reshape(M * NH * NP), inv_temp, eps_base_t)
    return probs.reshape(B, L, NH, NP)


# speedup vs baseline: 1.0991x; 1.0001x over previous
import functools

import jax
import jax.numpy as jnp
from jax import lax
from jax.experimental import pallas as pl
from jax.experimental.pallas import tpu as pltpu
from jax.experimental.pallas import tpu_sc as plsc

B, L, H_DIM = 2, 4096, 2048
NH, NP = 16, 5
MIN_ACTIVE = 2
FLOOR_START = 0.05

M = B * L
N = 2 * H_DIM
TM = 1024
TN = 2048

NUM_WORKERS = 32
TOK_W = M // NUM_WORKERS
CH = TOK_W * NH * NP


def _mlp_kernel(x_ref, w1_ref, b1_ref, w2_ref, b2_ref, out_ref):
    n = pl.program_id(1)
    h = (
        jnp.dot(
            x_ref[...].astype(jnp.bfloat16),
            w1_ref[...],
            preferred_element_type=jnp.float32,
        )
        + b1_ref[...]
    )
    h = 0.5 * h * (1.0 + jax.lax.erf(h * 0.7071067811865476))
    contrib = jnp.dot(
        h.astype(jnp.bfloat16), w2_ref[...], preferred_element_type=jnp.float32
    )

    @pl.when(n == 0)
    def _():
        out_ref[...] = contrib + b2_ref[...]

    @pl.when(n != 0)
    def _():
        out_ref[...] += contrib


def _sc_epilogue(logits_hbm, invt_hbm, epsb_hbm, out_hbm,
                 in_v, out_v, invt_v, epsb_v):
    wid = lax.axis_index("s") * 2 + lax.axis_index("c")
    base = wid * CH
    pltpu.sync_copy(logits_hbm.at[pl.ds(base, CH)], in_v)
    pltpu.sync_copy(invt_hbm, invt_v)
    pltpu.sync_copy(epsb_hbm, epsb_v)
    invt = invt_v[...]
    eb = [epsb_v[p, :] for p in range(NP)]
    lane5 = lax.iota(jnp.int32, 16) * 5

    def one_token(tbase):
        idx = [lane5 + (tbase + p) for p in range(NP)]
        z = [plsc.load_gather(in_v, [idx[p]]) * invt for p in range(NP)]
        zmax = jnp.maximum(jnp.maximum(jnp.maximum(z[0], z[1]),
                                       jnp.maximum(z[2], z[3])), z[4])
        e = [jnp.exp(z[p] - zmax) for p in range(NP)]
        s = (e[0] + e[1]) + (e[2] + e[3]) + e[4]
        r = 1.0 / s
        p_ = [e[p] * r for p in range(NP)]
        eps_scale = FLOOR_START * (1.0 - r)
        eps = [eps_scale * eb[p] for p in range(NP)]
        eps_sum = ((eps[0] + eps[1]) + (eps[2] + eps[3])) + eps[4]
        keep = 1.0 - eps_sum
        q = [jnp.minimum(jnp.maximum(p_[p] * keep + eps[p], 1e-9), 1.0)
             for p in range(NP)]
        cnt = sum((q[p] > 1e-6).astype(jnp.int32) for p in range(NP))
        insufficient = cnt < MIN_ACTIVE
        best1 = q[0]
        idx1 = jnp.zeros((16,), jnp.int32)
        for p in range(1, NP):
            better = q[p] > best1
            best1 = jnp.where(better, q[p], best1)
            idx1 = jnp.where(better, p, idx1)
        best2 = jnp.full((16,), -1.0, jnp.float32)
        idx2 = jnp.zeros((16,), jnp.int32)
        for p in range(NP):
            cand = jnp.where(idx1 == p, -1.0, q[p])
            better = cand > best2
            best2 = jnp.where(better, cand, best2)
            idx2 = jnp.where(better, p, idx2)
        for p in range(NP):
            on_top = jnp.logical_or(idx1 == p, idx2 == p)
            uni = jnp.where(on_top, 1.0 / MIN_ACTIVE, 0.0)
            res = jnp.where(insufficient, uni, q[p])
            plsc.store_scatter(out_v, [idx[p]], res)

    def body(t, carry):
        one_token(t * (2 * NH * NP))
        one_token(t * (2 * NH * NP) + NH * NP)
        return carry

    lax.fori_loop(0, TOK_W // 2, body, None)
    pltpu.sync_copy(out_v, out_hbm.at[pl.ds(base, CH)])


@jax.jit
def kernel(x, W1, b1, W2, b2, gate_log_temp, gate_eps_logit):
    x2 = x.reshape(M, H_DIM)
    W1b = W1.astype(jnp.bfloat16)
    W2b = W2.astype(jnp.bfloat16)
    logits = pl.pallas_call(
        _mlp_kernel,
        grid=(M // TM, N // TN),
        in_specs=[
            pl.BlockSpec((TM, H_DIM), lambda m, n: (m, 0)),
            pl.BlockSpec((H_DIM, TN), lambda m, n: (0, n)),
            pl.BlockSpec((1, TN), lambda m, n: (0, n)),
            pl.BlockSpec((TN, NH * NP), lambda m, n: (n, 0)),
            pl.BlockSpec((1, NH * NP), lambda m, n: (0, 0)),
        ],
        out_specs=pl.BlockSpec((TM, NH * NP), lambda m, n: (m, 0)),
        out_shape=jax.ShapeDtypeStruct((M, NH * NP), jnp.float32),
        compiler_params=pltpu.CompilerParams(
            dimension_semantics=("parallel", "arbitrary")
        ),
    )(x2, W1b, b1.reshape(1, N), W2b, b2.reshape(1, NH * NP))

    inv_temp = jnp.exp(-gate_log_temp)
    eps_base_t = jax.nn.sigmoid(gate_eps_logit).T

    mesh = plsc.VectorSubcoreMesh(core_axis_name="c", subcore_axis_name="s")
    sc_epi = functools.partial(
        pl.kernel,
        mesh=mesh,
        compiler_params=pltpu.CompilerParams(needs_layout_passes=False),
        out_type=jax.ShapeDtypeStruct((M * NH * NP,), jnp.float32),
        scratch_types=[
            pltpu.VMEM((CH,), jnp.float32),
            pltpu.VMEM((CH,), jnp.float32),
            pltpu.VMEM((NH,), jnp.float32),
            pltpu.VMEM((NP, NH), jnp.float32),
        ],
    )(_sc_epilogue)
    probs = sc_epi(logits.reshape(M * NH * NP), inv_temp, eps_base_t)
    return probs.reshape(B, L, NH, NP)
